# baseline (speedup 1.0000x reference)
import jax
import jax.numpy as jnp
from jax import lax
from jax.experimental import pallas as pl
from jax.experimental.pallas import tpu as pltpu
from jax.experimental.pallas import tpu_sc as plsc

_NC = 2
_NS = 16
_NW = _NC * _NS
_L = 16

_D = 64
_PAIR = 2 * _D
_CHUNK = 128


def _rsqrt(x):
    i = plsc.bitcast(x, jnp.int32)
    i = jnp.int32(0x5F3759DF) - lax.shift_right_logical(i, 1)
    y = plsc.bitcast(i, jnp.float32)
    for _ in range(3):
        y = y * (1.5 - 0.5 * x * y * y)
    return y


def _body(users, pos_items, neg_items, relations, ent2, rel2,
          out_pos, out_neg,
          idx_u, idx_p, idx_n, idx_r,
          hidx_u, hidx_p, hidx_n, hidx_r,
          rows_u, rows_p, rows_n, rows_r,
          outp_v, outn_v, sem):
    wid = lax.axis_index("s") * _NC + lax.axis_index("c")
    per_w = out_pos.shape[0] // _NW
    n_chunks = per_w // _CHUNK
    lane_iota = lax.iota(jnp.int32, _L)

    for c in range(n_chunks):
        base = wid * per_w + c * _CHUNK
        pltpu.sync_copy(users.at[pl.ds(base, _CHUNK)], idx_u)
        pltpu.sync_copy(pos_items.at[pl.ds(base, _CHUNK)], idx_p)
        pltpu.sync_copy(neg_items.at[pl.ds(base, _CHUNK)], idx_n)
        pltpu.sync_copy(relations.at[pl.ds(base, _CHUNK)], idx_r)

        def halve(g, _):
            sl = pl.ds(g * _L, _L)
            hidx_u[sl] = lax.shift_right_logical(idx_u[sl], 1)
            hidx_p[sl] = lax.shift_right_logical(idx_p[sl], 1)
            hidx_n[sl] = lax.shift_right_logical(idx_n[sl], 1)
            hidx_r[sl] = lax.shift_right_logical(idx_r[sl], 1)
            return 0

        lax.fori_loop(0, _CHUNK // _L, halve, 0)

        cp_u = pltpu.async_copy(ent2.at[hidx_u], rows_u, sem)
        cp_p = pltpu.async_copy(ent2.at[hidx_p], rows_p, sem)
        cp_n = pltpu.async_copy(ent2.at[hidx_n], rows_n, sem)
        cp_r = pltpu.async_copy(rel2.at[hidx_r], rows_r, sem)
        cp_u.wait()
        cp_p.wait()
        cp_n.wait()
        cp_r.wait()

        def group(g, _):
            gsl = pl.ds(g * _L, _L)
            rowv = lane_iota + g * _L
            one = jnp.int32(1)
            cb_u = lax.shift_left((idx_u[gsl] & one), 6)
            cb_p = lax.shift_left((idx_p[gsl] & one), 6)
            cb_n = lax.shift_left((idx_n[gsl] & one), 6)
            cb_r = lax.shift_left((idx_r[gsl] & one), 6)
            accp = jnp.zeros((_L,), jnp.float32)
            accn = jnp.zeros((_L,), jnp.float32)
            for d in range(_D):
                u = plsc.load_gather(rows_u, [rowv, cb_u + d])
                r = plsc.load_gather(rows_r, [rowv, cb_r + d])
                p = plsc.load_gather(rows_p, [rowv, cb_p + d])
                n = plsc.load_gather(rows_n, [rowv, cb_n + d])
                t = u + r
                dp = t - p
                dn = t - n
                accp = accp + dp * dp
                accn = accn + dn * dn
            outp_v[gsl] = -(accp * _rsqrt(jnp.maximum(accp, 1e-30)))
            outn_v[gsl] = -(accn * _rsqrt(jnp.maximum(accn, 1e-30)))
            return 0

        lax.fori_loop(0, _CHUNK // _L, group, 0)

        pltpu.sync_copy(outp_v, out_pos.at[pl.ds(base, _CHUNK)])
        pltpu.sync_copy(outn_v, out_neg.at[pl.ds(base, _CHUNK)])


def kernel(users, pos_items, neg_items, relations, ent_emb, rel_emb):
    B = users.shape[0]
    users = users.astype(jnp.int32)
    pos_items = pos_items.astype(jnp.int32)
    neg_items = neg_items.astype(jnp.int32)
    relations = relations.astype(jnp.int32)
    ent2 = ent_emb.reshape(ent_emb.shape[0] // 2, _PAIR)
    rel2 = rel_emb.reshape(rel_emb.shape[0] // 2, _PAIR)

    run = pl.kernel(
        _body,
        out_type=(
            jax.ShapeDtypeStruct((B,), jnp.float32),
            jax.ShapeDtypeStruct((B,), jnp.float32),
        ),
        mesh=plsc.VectorSubcoreMesh(
            core_axis_name="c", subcore_axis_name="s",
            num_cores=_NC, num_subcores=_NS,
        ),
        compiler_params=pltpu.CompilerParams(
            needs_layout_passes=False, use_tc_tiling_on_sc=True,
        ),
        scratch_types=[
            pltpu.VMEM((_CHUNK,), jnp.int32),
            pltpu.VMEM((_CHUNK,), jnp.int32),
            pltpu.VMEM((_CHUNK,), jnp.int32),
            pltpu.VMEM((_CHUNK,), jnp.int32),
            pltpu.VMEM((_CHUNK,), jnp.int32),
            pltpu.VMEM((_CHUNK,), jnp.int32),
            pltpu.VMEM((_CHUNK,), jnp.int32),
            pltpu.VMEM((_CHUNK,), jnp.int32),
            pltpu.VMEM((_CHUNK, _PAIR), jnp.float32),
            pltpu.VMEM((_CHUNK, _PAIR), jnp.float32),
            pltpu.VMEM((_CHUNK, _PAIR), jnp.float32),
            pltpu.VMEM((_CHUNK, _PAIR), jnp.float32),
            pltpu.VMEM((_CHUNK,), jnp.float32),
            pltpu.VMEM((_CHUNK,), jnp.float32),
            pltpu.SemaphoreType.DMA,
        ],
    )
    return run(users, pos_items, neg_items, relations, ent2, rel2)


# speedup vs baseline: 1.0006x; 1.0006x over previous
import jax
import jax.numpy as jnp
from jax import lax
from jax.experimental import pallas as pl
from jax.experimental.pallas import tpu as pltpu
from jax.experimental.pallas import tpu_sc as plsc

_NC = 2
_NS = 16
_NW = _NC * _NS
_L = 16

_D = 64
_PAIR = 2 * _D
_CHUNK = 128


def _rsqrt(x):
    i = plsc.bitcast(x, jnp.int32)
    i = jnp.int32(0x5F3759DF) - lax.shift_right_logical(i, 1)
    y = plsc.bitcast(i, jnp.float32)
    for _ in range(3):
        y = y * (1.5 - 0.5 * x * y * y)
    return y


def _body(users, pos_items, neg_items, relations, ent2, rel2,
          out_pos, out_neg,
          idx_u, idx_p, idx_n, idx_r,
          hidx_u, hidx_p, hidx_n, hidx_r,
          rows_u, rows_p, rows_n, rows_r,
          outp_v, outn_v, sem):
    wid = lax.axis_index("s") * _NC + lax.axis_index("c")
    per_w = out_pos.shape[0] // _NW
    n_chunks = per_w // _CHUNK
    lane_iota = lax.iota(jnp.int32, _L)

    for c in range(n_chunks):
        base = wid * per_w + c * _CHUNK
        pltpu.sync_copy(users.at[pl.ds(base, _CHUNK)], idx_u)
        pltpu.sync_copy(pos_items.at[pl.ds(base, _CHUNK)], idx_p)
        pltpu.sync_copy(neg_items.at[pl.ds(base, _CHUNK)], idx_n)
        pltpu.sync_copy(relations.at[pl.ds(base, _CHUNK)], idx_r)

        def halve(g, _):
            sl = pl.ds(g * _L, _L)
            hidx_u[sl] = lax.shift_right_logical(idx_u[sl], 1)
            hidx_p[sl] = lax.shift_right_logical(idx_p[sl], 1)
            hidx_n[sl] = lax.shift_right_logical(idx_n[sl], 1)
            hidx_r[sl] = lax.shift_right_logical(idx_r[sl], 1)
            return 0

        lax.fori_loop(0, _CHUNK // _L, halve, 0)

        cp_u = pltpu.async_copy(ent2.at[hidx_u], rows_u, sem)
        cp_p = pltpu.async_copy(ent2.at[hidx_p], rows_p, sem)
        cp_n = pltpu.async_copy(ent2.at[hidx_n], rows_n, sem)
        cp_r = pltpu.async_copy(rel2.at[hidx_r], rows_r, sem)
        cp_u.wait()
        cp_p.wait()
        cp_n.wait()
        cp_r.wait()

        def group(g, _):
            gsl = pl.ds(g * _L, _L)
            rowv = lane_iota + g * _L
            one = jnp.int32(1)
            cb_u = lax.shift_left(idx_u[gsl] & one, 6)
            cb_p = lax.shift_left(idx_p[gsl] & one, 6)
            cb_n = lax.shift_left(idx_n[gsl] & one, 6)
            cb_r = lax.shift_left(idx_r[gsl] & one, 6)
            accp = jnp.zeros((_L,), jnp.float32)
            accn = jnp.zeros((_L,), jnp.float32)
            for d in range(_D):
                u = plsc.load_gather(rows_u, [rowv, cb_u + d])
                r = plsc.load_gather(rows_r, [rowv, cb_r + d])
                p = plsc.load_gather(rows_p, [rowv, cb_p + d])
                n = plsc.load_gather(rows_n, [rowv, cb_n + d])
                t = u + r
                dp = t - p
                dn = t - n
                accp = accp + dp * dp
                accn = accn + dn * dn
            outp_v[gsl] = -(accp * _rsqrt(jnp.maximum(accp, 1e-30)))
            outn_v[gsl] = -(accn * _rsqrt(jnp.maximum(accn, 1e-30)))
            return 0

        lax.fori_loop(0, _CHUNK // _L, group, 0)

        pltpu.sync_copy(outp_v, out_pos.at[pl.ds(base, _CHUNK)])
        pltpu.sync_copy(outn_v, out_neg.at[pl.ds(base, _CHUNK)])


def kernel(users, pos_items, neg_items, relations, ent_emb, rel_emb):
    B = users.shape[0]
    users = users.astype(jnp.int32)
    pos_items = pos_items.astype(jnp.int32)
    neg_items = neg_items.astype(jnp.int32)
    relations = relations.astype(jnp.int32)
    ent2 = ent_emb.reshape(ent_emb.shape[0] // 2, _PAIR)
    rel2 = rel_emb.reshape(rel_emb.shape[0] // 2, _PAIR)

    run = pl.kernel(
        _body,
        out_type=(
            jax.ShapeDtypeStruct((B,), jnp.float32),
            jax.ShapeDtypeStruct((B,), jnp.float32),
        ),
        mesh=plsc.VectorSubcoreMesh(
            core_axis_name="c", subcore_axis_name="s",
            num_cores=_NC, num_subcores=_NS,
        ),
        compiler_params=pltpu.CompilerParams(
            needs_layout_passes=False, use_tc_tiling_on_sc=False,
        ),
        scratch_types=[
            pltpu.VMEM((_CHUNK,), jnp.int32),
            pltpu.VMEM((_CHUNK,), jnp.int32),
            pltpu.VMEM((_CHUNK,), jnp.int32),
            pltpu.VMEM((_CHUNK,), jnp.int32),
            pltpu.VMEM((_CHUNK,), jnp.int32),
            pltpu.VMEM((_CHUNK,), jnp.int32),
            pltpu.VMEM((_CHUNK,), jnp.int32),
            pltpu.VMEM((_CHUNK,), jnp.int32),
            pltpu.VMEM((_CHUNK, _PAIR), jnp.float32),
            pltpu.VMEM((_CHUNK, _PAIR), jnp.float32),
            pltpu.VMEM((_CHUNK, _PAIR), jnp.float32),
            pltpu.VMEM((_CHUNK, _PAIR), jnp.float32),
            pltpu.VMEM((_CHUNK,), jnp.float32),
            pltpu.VMEM((_CHUNK,), jnp.float32),
            pltpu.SemaphoreType.DMA,
        ],
    )
    return run(users, pos_items, neg_items, relations, ent2, rel2)


# speedup vs baseline: 1.7242x; 1.7231x over previous
import jax
import jax.numpy as jnp
from jax import lax
from jax.experimental import pallas as pl
from jax.experimental.pallas import tpu as pltpu
from jax.experimental.pallas import tpu_sc as plsc

_NC = 2
_NS = 16
_NW = _NC * _NS
_L = 16

_D = 64
_CHUNK = 128
_FIRE = 8


def _rsqrt(x):
    i = plsc.bitcast(x, jnp.int32)
    i = jnp.int32(0x5F3759DF) - lax.shift_right_logical(i, 1)
    y = plsc.bitcast(i, jnp.float32)
    for _ in range(3):
        y = y * (1.5 - 0.5 * x * y * y)
    return y


def _body(users, pos_items, neg_items, relations, ent_emb, rel_emb,
          out_pos, out_neg,
          idx_u, idx_p, idx_n, idx_r,
          rows_u, rows_p, rows_n, rows_r,
          outp_v, outn_v, sem):
    wid = lax.axis_index("s") * _NC + lax.axis_index("c")
    per_w = out_pos.shape[0] // _NW
    n_chunks = per_w // _CHUNK
    lane_iota = lax.iota(jnp.int32, _L)

    for c in range(n_chunks):
        base = wid * per_w + c * _CHUNK
        pltpu.sync_copy(users.at[pl.ds(base, _CHUNK)], idx_u)
        pltpu.sync_copy(pos_items.at[pl.ds(base, _CHUNK)], idx_p)
        pltpu.sync_copy(neg_items.at[pl.ds(base, _CHUNK)], idx_n)
        pltpu.sync_copy(relations.at[pl.ds(base, _CHUNK)], idx_r)

        def fetch(s, _):
            e0 = s * _L
            gsl = pl.ds(e0, _L)
            vu = idx_u[gsl]
            vp = idx_p[gsl]
            vn = idx_n[gsl]
            vr = idx_r[gsl]
            cps = []
            for j in range(_L):
                e = e0 + j
                cps.append(pltpu.async_copy(
                    ent_emb.at[pl.ds(vu[j], 1), :],
                    rows_u.at[pl.ds(e, 1), :], sem))
                cps.append(pltpu.async_copy(
                    ent_emb.at[pl.ds(vp[j], 1), :],
                    rows_p.at[pl.ds(e, 1), :], sem))
                cps.append(pltpu.async_copy(
                    ent_emb.at[pl.ds(vn[j], 1), :],
                    rows_n.at[pl.ds(e, 1), :], sem))
                cps.append(pltpu.async_copy(
                    rel_emb.at[pl.ds(vr[j], 1), :],
                    rows_r.at[pl.ds(e, 1), :], sem))
            for cp in cps:
                cp.wait()
            return 0

        lax.fori_loop(0, _CHUNK // _L, fetch, 0)

        def group(g, _):
            gsl = pl.ds(g * _L, _L)
            resp = jnp.zeros((_L,), jnp.float32)
            resn = jnp.zeros((_L,), jnp.float32)
            for j in range(_L):
                e = g * _L + j
                accp = jnp.zeros((_L,), jnp.float32)
                accn = jnp.zeros((_L,), jnp.float32)
                for k in range(_D // _L):
                    sl = pl.ds(k * _L, _L)
                    u = rows_u[e, sl]
                    r = rows_r[e, sl]
                    p = rows_p[e, sl]
                    n = rows_n[e, sl]
                    t = u + r
                    dp = t - p
                    dn = t - n
                    accp = accp + dp * dp
                    accn = accn + dn * dn
                lane = lane_iota == j
                resp = jnp.where(lane, jnp.sum(accp), resp)
                resn = jnp.where(lane, jnp.sum(accn), resn)
            outp_v[gsl] = -(resp * _rsqrt(jnp.maximum(resp, 1e-30)))
            outn_v[gsl] = -(resn * _rsqrt(jnp.maximum(resn, 1e-30)))
            return 0

        lax.fori_loop(0, _CHUNK // _L, group, 0)

        pltpu.sync_copy(outp_v, out_pos.at[pl.ds(base, _CHUNK)])
        pltpu.sync_copy(outn_v, out_neg.at[pl.ds(base, _CHUNK)])


def kernel(users, pos_items, neg_items, relations, ent_emb, rel_emb):
    B = users.shape[0]
    users = users.astype(jnp.int32)
    pos_items = pos_items.astype(jnp.int32)
    neg_items = neg_items.astype(jnp.int32)
    relations = relations.astype(jnp.int32)

    run = pl.kernel(
        _body,
        out_type=(
            jax.ShapeDtypeStruct((B,), jnp.float32),
            jax.ShapeDtypeStruct((B,), jnp.float32),
        ),
        mesh=plsc.VectorSubcoreMesh(
            core_axis_name="c", subcore_axis_name="s",
            num_cores=_NC, num_subcores=_NS,
        ),
        compiler_params=pltpu.CompilerParams(
            needs_layout_passes=False, use_tc_tiling_on_sc=True,
        ),
        scratch_types=[
            pltpu.VMEM((_CHUNK,), jnp.int32),
            pltpu.VMEM((_CHUNK,), jnp.int32),
            pltpu.VMEM((_CHUNK,), jnp.int32),
            pltpu.VMEM((_CHUNK,), jnp.int32),
            pltpu.VMEM((_CHUNK, _D), jnp.float32),
            pltpu.VMEM((_CHUNK, _D), jnp.float32),
            pltpu.VMEM((_CHUNK, _D), jnp.float32),
            pltpu.VMEM((_CHUNK, _D), jnp.float32),
            pltpu.VMEM((_CHUNK,), jnp.float32),
            pltpu.VMEM((_CHUNK,), jnp.float32),
            pltpu.SemaphoreType.DMA,
        ],
    )
    return run(users, pos_items, neg_items, relations, ent_emb, rel_emb)


# speedup vs baseline: 1.7846x; 1.0351x over previous
import jax
import jax.numpy as jnp
from jax import lax
from jax.experimental import pallas as pl
from jax.experimental.pallas import tpu as pltpu
from jax.experimental.pallas import tpu_sc as plsc

_NC = 2
_NS = 16
_NW = _NC * _NS
_L = 16

_D = 64
_CHUNK = 64


def _rsqrt(x):
    i = plsc.bitcast(x, jnp.int32)
    i = jnp.int32(0x5F3759DF) - lax.shift_right_logical(i, 1)
    y = plsc.bitcast(i, jnp.float32)
    for _ in range(3):
        y = y * (1.5 - 0.5 * x * y * y)
    return y


def _body(users, pos_items, neg_items, relations, ent_emb, rel_emb,
          out_pos, out_neg,
          idx_u, idx_p, idx_n, idx_r,
          ru0, rp0, rn0, rr0, ru1, rp1, rn1, rr1,
          outp_v, outn_v, sem0, sem1):
    wid = lax.axis_index("s") * _NC + lax.axis_index("c")
    per_w = out_pos.shape[0] // _NW
    n_chunks = per_w // _CHUNK
    wbase = wid * per_w
    lane_iota = lax.iota(jnp.int32, _L)

    bufs = [(ru0, rp0, rn0, rr0, sem0), (ru1, rp1, rn1, rr1, sem1)]

    def fire(c, bset):
        ru, rp, rn, rr, sem = bset
        cbase = wbase + c * _CHUNK
        pltpu.sync_copy(users.at[pl.ds(cbase, _CHUNK)], idx_u)
        pltpu.sync_copy(pos_items.at[pl.ds(cbase, _CHUNK)], idx_p)
        pltpu.sync_copy(neg_items.at[pl.ds(cbase, _CHUNK)], idx_n)
        pltpu.sync_copy(relations.at[pl.ds(cbase, _CHUNK)], idx_r)

        def fgroup(g, _):
            e0 = g * _L
            gsl = pl.ds(e0, _L)
            vu = idx_u[gsl]
            vp = idx_p[gsl]
            vn = idx_n[gsl]
            vr = idx_r[gsl]
            for j in range(_L):
                e = e0 + j
                pltpu.async_copy(ent_emb.at[pl.ds(vu[j], 1), :],
                                 ru.at[pl.ds(e, 1), :], sem)
                pltpu.async_copy(ent_emb.at[pl.ds(vp[j], 1), :],
                                 rp.at[pl.ds(e, 1), :], sem)
                pltpu.async_copy(ent_emb.at[pl.ds(vn[j], 1), :],
                                 rn.at[pl.ds(e, 1), :], sem)
                pltpu.async_copy(rel_emb.at[pl.ds(vr[j], 1), :],
                                 rr.at[pl.ds(e, 1), :], sem)
            return 0

        lax.fori_loop(0, _CHUNK // _L, fgroup, 0)

    def drain(bset):
        ru, rp, rn, rr, sem = bset
        for dst in (ru, rp, rn, rr):
            pltpu.make_async_copy(ent_emb.at[pl.ds(0, _CHUNK), :], dst,
                                  sem).wait()

    def score(c, bset):
        ru, rp, rn, rr, _ = bset

        def group(g, _):
            resp = jnp.zeros((_L,), jnp.float32)
            resn = jnp.zeros((_L,), jnp.float32)
            for j in range(_L):
                e = g * _L + j
                accp = jnp.zeros((_L,), jnp.float32)
                accn = jnp.zeros((_L,), jnp.float32)
                for k in range(_D // _L):
                    sl = pl.ds(k * _L, _L)
                    u = ru[e, sl]
                    r = rr[e, sl]
                    p = rp[e, sl]
                    n = rn[e, sl]
                    t = u + r
                    dp = t - p
                    dn = t - n
                    accp = accp + dp * dp
                    accn = accn + dn * dn
                lane = lane_iota == j
                resp = jnp.where(lane, jnp.sum(accp), resp)
                resn = jnp.where(lane, jnp.sum(accn), resn)
            gsl = pl.ds(g * _L, _L)
            outp_v[gsl] = -(resp * _rsqrt(jnp.maximum(resp, 1e-30)))
            outn_v[gsl] = -(resn * _rsqrt(jnp.maximum(resn, 1e-30)))
            return 0

        lax.fori_loop(0, _CHUNK // _L, group, 0)
        cbase = wbase + c * _CHUNK
        pltpu.sync_copy(outp_v, out_pos.at[pl.ds(cbase, _CHUNK)])
        pltpu.sync_copy(outn_v, out_neg.at[pl.ds(cbase, _CHUNK)])

    fire(0, bufs[0])
    for c in range(n_chunks):
        if c + 1 < n_chunks:
            fire(c + 1, bufs[(c + 1) % 2])
        drain(bufs[c % 2])
        score(c, bufs[c % 2])


def kernel(users, pos_items, neg_items, relations, ent_emb, rel_emb):
    B = users.shape[0]
    users = users.astype(jnp.int32)
    pos_items = pos_items.astype(jnp.int32)
    neg_items = neg_items.astype(jnp.int32)
    relations = relations.astype(jnp.int32)
    per_w = B // _NW

    run = pl.kernel(
        _body,
        out_type=(
            jax.ShapeDtypeStruct((B,), jnp.float32),
            jax.ShapeDtypeStruct((B,), jnp.float32),
        ),
        mesh=plsc.VectorSubcoreMesh(
            core_axis_name="c", subcore_axis_name="s",
            num_cores=_NC, num_subcores=_NS,
        ),
        compiler_params=pltpu.CompilerParams(
            needs_layout_passes=False, use_tc_tiling_on_sc=True,
        ),
        scratch_types=[
            pltpu.VMEM((_CHUNK,), jnp.int32),
            pltpu.VMEM((_CHUNK,), jnp.int32),
            pltpu.VMEM((_CHUNK,), jnp.int32),
            pltpu.VMEM((_CHUNK,), jnp.int32),
            pltpu.VMEM((_CHUNK, _D), jnp.float32),
            pltpu.VMEM((_CHUNK, _D), jnp.float32),
            pltpu.VMEM((_CHUNK, _D), jnp.float32),
            pltpu.VMEM((_CHUNK, _D), jnp.float32),
            pltpu.VMEM((_CHUNK, _D), jnp.float32),
            pltpu.VMEM((_CHUNK, _D), jnp.float32),
            pltpu.VMEM((_CHUNK, _D), jnp.float32),
            pltpu.VMEM((_CHUNK, _D), jnp.float32),
            pltpu.VMEM((_CHUNK,), jnp.float32),
            pltpu.VMEM((_CHUNK,), jnp.float32),
            pltpu.SemaphoreType.DMA,
            pltpu.SemaphoreType.DMA,
        ],
    )
    return run(users, pos_items, neg_items, relations, ent_emb, rel_emb)
